# baseline (speedup 1.0000x reference)
import functools

import jax
import jax.numpy as jnp
from jax import lax
from jax.experimental import pallas as pl
from jax.experimental.pallas import tpu as pltpu
from jax.experimental.pallas import tpu_sc as plsc

D_MODEL = 64
CHUNK = 512


def _make_gather(n_rows: int, d: int):
    info = plsc.get_sparse_core_info()
    nc, ns = info.num_cores, info.num_subcores
    nw = nc * ns
    assert n_rows % (nw * CHUNK) == 0
    rows_per_w = n_rows // nw
    n_iters = rows_per_w // CHUNK

    mesh = plsc.VectorSubcoreMesh(core_axis_name="c", subcore_axis_name="s")

    @functools.partial(
        pl.kernel,
        out_type=jax.ShapeDtypeStruct((n_rows, d), jnp.float32),
        mesh=mesh,
        scratch_types=[
            pltpu.VMEM((CHUNK,), jnp.int32),
            pltpu.VMEM((CHUNK, d), jnp.float32),
            pltpu.SemaphoreType.DMA,
        ],
        compiler_params=pltpu.CompilerParams(use_tc_tiling_on_sc=False),
    )
    def gather_kernel(table_hbm, idx_hbm, out_hbm, idx_v, rows_v, sem):
        wid = lax.axis_index("s") * nc + lax.axis_index("c")
        base = wid * rows_per_w

        def body(i, carry):
            off = base + i * CHUNK
            pltpu.sync_copy(idx_hbm.at[pl.ds(off, CHUNK)], idx_v)
            pltpu.async_copy(table_hbm.at[idx_v], rows_v, sem).wait()
            pltpu.sync_copy(rows_v, out_hbm.at[pl.ds(off, CHUNK)])
            return carry

        lax.fori_loop(0, n_iters, body, 0)

    return gather_kernel


def kernel(x, table):
    b, s = x.shape
    idx = x.reshape(-1).astype(jnp.int32)
    out = _make_gather(b * s, D_MODEL)(table, idx)
    return out.reshape(b, s, D_MODEL)


# speedup vs baseline: 1.0435x; 1.0435x over previous
import functools

import jax
import jax.numpy as jnp
from jax import lax
from jax.experimental import pallas as pl
from jax.experimental.pallas import tpu as pltpu
from jax.experimental.pallas import tpu_sc as plsc

D_MODEL = 64
CHUNK = 512


def _make_gather(n_rows: int, d: int):
    info = plsc.get_sparse_core_info()
    nc, ns = info.num_cores, info.num_subcores
    nw = nc * ns
    assert n_rows % (nw * CHUNK) == 0
    rows_per_w = n_rows // nw
    n_iters = rows_per_w // CHUNK
    assert n_iters % 2 == 0

    mesh = plsc.VectorSubcoreMesh(core_axis_name="c", subcore_axis_name="s")

    @functools.partial(
        pl.kernel,
        out_type=jax.ShapeDtypeStruct((n_rows, d), jnp.float32),
        mesh=mesh,
        scratch_types=[
            pltpu.VMEM((rows_per_w,), jnp.int32),
            pltpu.VMEM((2, CHUNK, d), jnp.float32),
            pltpu.SemaphoreType.DMA,
            pltpu.SemaphoreType.DMA,
        ],
        compiler_params=pltpu.CompilerParams(use_tc_tiling_on_sc=False),
    )
    def gather_kernel(table_hbm, idx_hbm, out_hbm, idx_all, rows_v, sem0, sem1):
        wid = lax.axis_index("s") * nc + lax.axis_index("c")
        base = wid * rows_per_w
        pltpu.sync_copy(idx_hbm.at[pl.ds(base, rows_per_w)], idx_all)

        def gather_copy(i, buf, sem):
            return pltpu.make_async_copy(
                table_hbm.at[idx_all.at[pl.ds(i * CHUNK, CHUNK)]],
                rows_v.at[buf],
                sem,
            )

        gather_copy(0, 0, sem0).start()
        gather_copy(1, 1, sem1).start()

        def body(g, carry):
            i0 = 2 * g

            gather_copy(i0, 0, sem0).wait()
            pltpu.sync_copy(rows_v.at[0], out_hbm.at[pl.ds(base + i0 * CHUNK, CHUNK)])

            @pl.when(i0 + 2 < n_iters)
            def _():
                gather_copy(i0 + 2, 0, sem0).start()

            i1 = i0 + 1
            gather_copy(i1, 1, sem1).wait()
            pltpu.sync_copy(rows_v.at[1], out_hbm.at[pl.ds(base + i1 * CHUNK, CHUNK)])

            @pl.when(i1 + 2 < n_iters)
            def _():
                gather_copy(i1 + 2, 1, sem1).start()

            return carry

        lax.fori_loop(0, n_iters // 2, body, 0)

    return gather_kernel


def kernel(x, table):
    b, s = x.shape
    idx = x.reshape(-1).astype(jnp.int32)
    out = _make_gather(b * s, D_MODEL)(table, idx)
    return out.reshape(b, s, D_MODEL)


# speedup vs baseline: 1.0471x; 1.0034x over previous
import functools

import jax
import jax.numpy as jnp
from jax import lax
from jax.experimental import pallas as pl
from jax.experimental.pallas import tpu as pltpu
from jax.experimental.pallas import tpu_sc as plsc

NBUF = 4


def _make_gather(batch: int, seq: int, d: int):
    info = plsc.get_sparse_core_info()
    nc, ns = info.num_cores, info.num_subcores
    nw = nc * ns
    assert batch % nw == 0
    rows_per_w = batch // nw
    assert rows_per_w % NBUF == 0

    mesh = plsc.VectorSubcoreMesh(core_axis_name="c", subcore_axis_name="s")

    @functools.partial(
        pl.kernel,
        out_type=jax.ShapeDtypeStruct((batch, seq, d), jnp.float32),
        mesh=mesh,
        scratch_types=[
            pltpu.VMEM((rows_per_w, seq), jnp.int32),
            pltpu.VMEM((NBUF, seq, d), jnp.float32),
        ]
        + [pltpu.SemaphoreType.DMA] * NBUF,
        compiler_params=pltpu.CompilerParams(use_tc_tiling_on_sc=False),
    )
    def gather_kernel(x_hbm, table_hbm, out_hbm, idx_v, rows_v, *sems):
        wid = lax.axis_index("s") * nc + lax.axis_index("c")
        base = wid * rows_per_w
        pltpu.sync_copy(x_hbm.at[pl.ds(base, rows_per_w)], idx_v)

        def gather_copy(i, buf):
            return pltpu.make_async_copy(
                table_hbm.at[idx_v.at[i]],
                rows_v.at[buf],
                sems[buf],
            )

        for b in range(NBUF):
            gather_copy(b, b).start()

        def body(g, carry):
            for b in range(NBUF):
                i = NBUF * g + b
                gather_copy(i, b).wait()
                pltpu.sync_copy(rows_v.at[b], out_hbm.at[base + i])

                @pl.when(i + NBUF < rows_per_w)
                def _():
                    gather_copy(i + NBUF, b).start()

            return carry

        lax.fori_loop(0, rows_per_w // NBUF, body, 0)

    return gather_kernel


def kernel(x, table):
    b, s = x.shape
    _, d = table.shape
    return _make_gather(b, s, d)(x.astype(jnp.int32), table)


# speedup vs baseline: 1.2754x; 1.2181x over previous
import functools

import jax
import jax.numpy as jnp
from jax import lax
from jax.experimental import pallas as pl
from jax.experimental.pallas import tpu as pltpu
from jax.experimental.pallas import tpu_sc as plsc

NBUF = 2
LANES = 128


def _make_gather(batch: int, seq: int, vocab: int):
    info = plsc.get_sparse_core_info()
    nc, ns = info.num_cores, info.num_subcores
    nw = nc * ns
    assert batch % nw == 0
    rows_per_w = batch // nw
    toks_per_w = rows_per_w * seq

    mesh = plsc.VectorSubcoreMesh(core_axis_name="c", subcore_axis_name="s")

    @functools.partial(
        pl.kernel,
        out_type=jax.ShapeDtypeStruct((batch * seq, LANES), jnp.float32),
        mesh=mesh,
        scratch_types=[
            pltpu.VMEM((toks_per_w,), jnp.int32),
            pltpu.VMEM((NBUF, seq, LANES), jnp.float32),
        ]
        + [pltpu.SemaphoreType.DMA] * NBUF,
    )
    def gather_kernel(idx_hbm, table_hbm, out_hbm, idx_v, rows_v, *sems):
        wid = lax.axis_index("s") * nc + lax.axis_index("c")
        base = wid * rows_per_w
        pltpu.sync_copy(idx_hbm.at[pl.ds(base * seq, toks_per_w)], idx_v)

        def gather_copy(i, buf):
            return pltpu.make_async_copy(
                table_hbm.at[idx_v.at[pl.ds(i * seq, seq)]],
                rows_v.at[buf],
                sems[buf],
            )

        for b in range(NBUF):
            gather_copy(b, b).start()

        def body(g, carry):
            for b in range(NBUF):
                i = NBUF * g + b
                gather_copy(i, b).wait()
                pltpu.sync_copy(
                    rows_v.at[b], out_hbm.at[pl.ds((base + i) * seq, seq)]
                )

                @pl.when(i + NBUF < rows_per_w)
                def _():
                    gather_copy(i + NBUF, b).start()

            return carry

        lax.fori_loop(0, rows_per_w // NBUF, body, 0)

    return gather_kernel


def kernel(x, table):
    b, s = x.shape
    v, d = table.shape
    idx = x.reshape(-1).astype(jnp.int32)
    table_p = jnp.pad(table, ((0, 0), (0, LANES - d)))
    out_p = _make_gather(b, s, v)(idx, table_p)
    return out_p[:, :d].reshape(b, s, d)
